# baseline (speedup 1.0000x reference)
import functools

import jax
import jax.numpy as jnp
from jax.experimental import pallas as pl

N = 2048
D = 512
H1 = 512
H2 = 512
NOUT = 128
HEADS = 4
DH = H1 // HEADS
BLK = 256
NEG = -1e9


def _proj_kernel(x_ref, nt_ref, wi_ref, wv_ref, wc_ref, h_ref):
    x = x_ref[...]
    t = nt_ref[...]
    h0 = jnp.dot(x, wi_ref[...], preferred_element_type=jnp.float32)
    h1 = jnp.dot(x, wv_ref[...], preferred_element_type=jnp.float32)
    h2 = jnp.dot(x, wc_ref[...], preferred_element_type=jnp.float32)
    h_ref[...] = jnp.where(t == 0, h0, jnp.where(t == 1, h1, h2))


def _attn_kernel(h_ref, asrc_ref, adst_ref, adj_c_ref, adj_d_ref, adj_l_ref,
                 fc1w_ref, fc1b_ref, fc2w_ref, fc2b_ref, out_ref):
    i = pl.program_id(0)
    h = h_ref[...]
    h_bf = h.astype(jnp.bfloat16)
    h_blk = h_ref[pl.ds(i * BLK, BLK), :]
    s_blk = jnp.dot(h_blk, asrc_ref[...], preferred_element_type=jnp.float32)
    dT = jax.lax.dot_general(adst_ref[...], h,
                             (((1,), (1,)), ((), ())),
                             preferred_element_type=jnp.float32)
    z_parts = []
    for t, adj_ref in enumerate((adj_c_ref, adj_d_ref, adj_l_ref)):
        adj = adj_ref[...]
        edge = adj > 0.0
        heads = []
        for hd in range(HEADS):
            col = t * HEADS + hd
            e = s_blk[:, col:col + 1] + dT[col:col + 1, :]
            e = jnp.where(e >= 0.0, e, 0.01 * e)
            e = jnp.where(edge, e, NEG)
            m = jnp.max(e, axis=1, keepdims=True)
            w = jnp.exp(e - m)
            z = jnp.sum(w, axis=1, keepdims=True)
            o = jnp.dot(w.astype(jnp.bfloat16), h_bf[:, hd * DH:(hd + 1) * DH],
                        preferred_element_type=jnp.float32)
            heads.append(o / z)
        z_parts.append(jnp.concatenate(heads, axis=1))
    z = z_parts[0] + z_parts[1] + z_parts[2]
    z = jnp.dot(z, fc1w_ref[...], preferred_element_type=jnp.float32) + fc1b_ref[...]
    z = jnp.where(z >= 0.0, z, 0.1 * z)
    z = jnp.dot(z, fc2w_ref[...], preferred_element_type=jnp.float32) + fc2b_ref[...]
    out_ref[...] = jnp.where(z >= 0.0, z, 0.1 * z)


@functools.partial(jax.jit, static_argnames=())
def kernel(x, node_types, adj_mat_control, adj_mat_data, adj_mat_call,
           W_inst, W_var, W_const, a_src, a_dst, fc1_w, fc1_b, fc2_w, fc2_b):
    nt = node_types.astype(jnp.int32).reshape(N, 1)
    h = pl.pallas_call(
        _proj_kernel,
        grid=(N // BLK,),
        in_specs=[
            pl.BlockSpec((BLK, D), lambda i: (i, 0)),
            pl.BlockSpec((BLK, 1), lambda i: (i, 0)),
            pl.BlockSpec((D, H1), lambda i: (0, 0)),
            pl.BlockSpec((D, H1), lambda i: (0, 0)),
            pl.BlockSpec((D, H1), lambda i: (0, 0)),
        ],
        out_specs=pl.BlockSpec((BLK, H1), lambda i: (i, 0)),
        out_shape=jax.ShapeDtypeStruct((N, H1), jnp.float32),
    )(x, nt, W_inst, W_var, W_const)

    ncol = 3 * HEADS
    hd_idx = jnp.arange(H1) // DH
    col = jnp.arange(ncol)
    sel = (hd_idx[:, None] == (col[None, :] % HEADS)).astype(jnp.float32)
    a_src_m = a_src.transpose(1, 2, 0).reshape(H1, 3)
    a_dst_m = a_dst.transpose(1, 2, 0).reshape(H1, 3)
    A_src = a_src_m[:, col // HEADS] * sel
    A_dst = a_dst_m[:, col // HEADS] * sel
    A_dst_T = A_dst.T

    out = pl.pallas_call(
        _attn_kernel,
        grid=(N // BLK,),
        in_specs=[
            pl.BlockSpec((N, H1), lambda i: (0, 0)),
            pl.BlockSpec((H1, ncol), lambda i: (0, 0)),
            pl.BlockSpec((ncol, H1), lambda i: (0, 0)),
            pl.BlockSpec((BLK, N), lambda i: (i, 0)),
            pl.BlockSpec((BLK, N), lambda i: (i, 0)),
            pl.BlockSpec((BLK, N), lambda i: (i, 0)),
            pl.BlockSpec((H1, H2), lambda i: (0, 0)),
            pl.BlockSpec((1, H2), lambda i: (0, 0)),
            pl.BlockSpec((H2, NOUT), lambda i: (0, 0)),
            pl.BlockSpec((1, NOUT), lambda i: (0, 0)),
        ],
        out_specs=pl.BlockSpec((BLK, NOUT), lambda i: (i, 0)),
        out_shape=jax.ShapeDtypeStruct((N, NOUT), jnp.float32),
    )(h, A_src, A_dst_T, adj_mat_control, adj_mat_data, adj_mat_call,
      fc1_w, fc1_b.reshape(1, H2), fc2_w, fc2_b.reshape(1, NOUT))
    return out


# speedup vs baseline: 1.2669x; 1.2669x over previous
import jax
import jax.numpy as jnp
from jax.experimental import pallas as pl

N = 2048
D = 512
H1 = 512
H2 = 512
NOUT = 128
HEADS = 4
DH = H1 // HEADS
BLK = 256
NCOL = 3 * HEADS


def _proj_kernel(x_ref, nt_ref, wi_ref, wv_ref, wc_ref, asrc_ref, adstT_ref,
                 h_ref, s_ref, dT_ref):
    x = x_ref[...]
    t = nt_ref[...]
    h0 = jnp.dot(x, wi_ref[...], preferred_element_type=jnp.float32)
    h1 = jnp.dot(x, wv_ref[...], preferred_element_type=jnp.float32)
    h2 = jnp.dot(x, wc_ref[...], preferred_element_type=jnp.float32)
    h = jnp.where(t == 0, h0, jnp.where(t == 1, h1, h2))
    h_ref[...] = h
    s_ref[...] = jnp.dot(h, asrc_ref[...], preferred_element_type=jnp.float32)
    dT_ref[...] = jax.lax.dot_general(adstT_ref[...], h, (((1,), (1,)), ((), ())),
                                      preferred_element_type=jnp.float32)


def _attn_kernel(h_ref, s_ref, dT_ref, adj_c_ref, adj_d_ref, adj_l_ref,
                 fc1w_ref, fc1b_ref, fc2w_ref, fc2b_ref, out_ref):
    h = h_ref[...]
    s_blk = s_ref[...]
    dT = dT_ref[...]
    z_parts = []
    for t, adj_ref in enumerate((adj_c_ref, adj_d_ref, adj_l_ref)):
        adj = adj_ref[...]
        heads = []
        for hd in range(HEADS):
            col = t * HEADS + hd
            e = s_blk[:, col:col + 1] + dT[col:col + 1, :]
            e = jnp.maximum(e, 0.01 * e)
            w = adj * jnp.exp(e) + 1e-35
            z = jnp.sum(w, axis=1, keepdims=True)
            o = jnp.dot(w, h[:, hd * DH:(hd + 1) * DH],
                        preferred_element_type=jnp.float32)
            heads.append(o / z)
        z_parts.append(jnp.concatenate(heads, axis=1))
    z = z_parts[0] + z_parts[1] + z_parts[2]
    z = jnp.dot(z, fc1w_ref[...], preferred_element_type=jnp.float32) + fc1b_ref[...]
    z = jnp.where(z >= 0.0, z, 0.1 * z)
    z = jnp.dot(z, fc2w_ref[...], preferred_element_type=jnp.float32) + fc2b_ref[...]
    out_ref[...] = jnp.where(z >= 0.0, z, 0.1 * z)


def kernel(x, node_types, adj_mat_control, adj_mat_data, adj_mat_call,
           W_inst, W_var, W_const, a_src, a_dst, fc1_w, fc1_b, fc2_w, fc2_b):
    nt = node_types.astype(jnp.int32).reshape(N, 1)

    hd_idx = jnp.arange(H1) // DH
    col = jnp.arange(NCOL)
    sel = (hd_idx[:, None] == (col[None, :] % HEADS)).astype(jnp.float32)
    a_src_m = a_src.transpose(1, 2, 0).reshape(H1, 3)
    a_dst_m = a_dst.transpose(1, 2, 0).reshape(H1, 3)
    A_src = a_src_m[:, col // HEADS] * sel
    A_dst_T = (a_dst_m[:, col // HEADS] * sel).T

    h, s, dT = pl.pallas_call(
        _proj_kernel,
        grid=(N // BLK,),
        in_specs=[
            pl.BlockSpec((BLK, D), lambda i: (i, 0)),
            pl.BlockSpec((BLK, 1), lambda i: (i, 0)),
            pl.BlockSpec((D, H1), lambda i: (0, 0)),
            pl.BlockSpec((D, H1), lambda i: (0, 0)),
            pl.BlockSpec((D, H1), lambda i: (0, 0)),
            pl.BlockSpec((H1, NCOL), lambda i: (0, 0)),
            pl.BlockSpec((NCOL, H1), lambda i: (0, 0)),
        ],
        out_specs=[
            pl.BlockSpec((BLK, H1), lambda i: (i, 0)),
            pl.BlockSpec((BLK, NCOL), lambda i: (i, 0)),
            pl.BlockSpec((NCOL, BLK), lambda i: (0, i)),
        ],
        out_shape=[
            jax.ShapeDtypeStruct((N, H1), jnp.float32),
            jax.ShapeDtypeStruct((N, NCOL), jnp.float32),
            jax.ShapeDtypeStruct((NCOL, N), jnp.float32),
        ],
    )(x, nt, W_inst, W_var, W_const, A_src, A_dst_T)

    out = pl.pallas_call(
        _attn_kernel,
        grid=(N // BLK,),
        in_specs=[
            pl.BlockSpec((N, H1), lambda i: (0, 0)),
            pl.BlockSpec((BLK, NCOL), lambda i: (i, 0)),
            pl.BlockSpec((NCOL, N), lambda i: (0, 0)),
            pl.BlockSpec((BLK, N), lambda i: (i, 0)),
            pl.BlockSpec((BLK, N), lambda i: (i, 0)),
            pl.BlockSpec((BLK, N), lambda i: (i, 0)),
            pl.BlockSpec((H1, H2), lambda i: (0, 0)),
            pl.BlockSpec((1, H2), lambda i: (0, 0)),
            pl.BlockSpec((H2, NOUT), lambda i: (0, 0)),
            pl.BlockSpec((1, NOUT), lambda i: (0, 0)),
        ],
        out_specs=pl.BlockSpec((BLK, NOUT), lambda i: (i, 0)),
        out_shape=jax.ShapeDtypeStruct((N, NOUT), jnp.float32),
    )(h, s, dT, adj_mat_control, adj_mat_data, adj_mat_call,
      fc1_w, fc1_b.reshape(1, H2), fc2_w, fc2_b.reshape(1, NOUT))
    return out


# speedup vs baseline: 1.7224x; 1.3596x over previous
import math

import jax
import jax.numpy as jnp
from jax.experimental import pallas as pl

N = 2048
D = 512
H1 = 512
H2 = 512
NOUT = 128
HEADS = 4
DH = H1 // HEADS
BLK = 256
NCOL = 3 * HEADS
EXT = DH + 128
LOG2E = math.log2(math.e)


def _proj_kernel(x_ref, nt_ref, wi_ref, wv_ref, wc_ref, asrc_ref, adstT_ref,
                 hext_ref, s_ref, dT_ref):
    x = x_ref[...]
    t = nt_ref[...]
    h0 = jnp.dot(x, wi_ref[...], preferred_element_type=jnp.float32)
    h1 = jnp.dot(x, wv_ref[...], preferred_element_type=jnp.float32)
    h2 = jnp.dot(x, wc_ref[...], preferred_element_type=jnp.float32)
    h = jnp.where(t == 0, h0, jnp.where(t == 1, h1, h2))
    s_ref[...] = jnp.dot(h, asrc_ref[...], preferred_element_type=jnp.float32)
    dT_ref[...] = jax.lax.dot_general(adstT_ref[...], h, (((1,), (1,)), ((), ())),
                                      preferred_element_type=jnp.float32)
    ones_col = (jax.lax.broadcasted_iota(jnp.int32, (BLK, 128), 1) == 0)
    ones_col = ones_col.astype(jnp.float32)
    parts = []
    for hd in range(HEADS):
        parts.append(h[:, hd * DH:(hd + 1) * DH])
        parts.append(ones_col)
    hext_ref[...] = jnp.concatenate(parts, axis=1).astype(jnp.bfloat16)


def _attn_kernel(hext_ref, s_ref, dT_ref, adj_c_ref, adj_d_ref, adj_l_ref,
                 fc1w_ref, fc1b_ref, fc2w_ref, fc2b_ref, out_ref):
    hext = hext_ref[...]
    s_blk = s_ref[...]
    dT = dT_ref[...]
    z_parts = []
    for t, adj_ref in enumerate((adj_c_ref, adj_d_ref, adj_l_ref)):
        adj = adj_ref[...]
        heads = []
        for hd in range(HEADS):
            col = t * HEADS + hd
            e = s_blk[:, col:col + 1] + dT[col:col + 1, :]
            e = jnp.maximum(e, 0.01 * e)
            w = adj * jnp.exp2(e) + 1e-35
            o_ext = jnp.dot(w.astype(jnp.bfloat16),
                            hext[:, hd * EXT:(hd + 1) * EXT],
                            preferred_element_type=jnp.float32)
            heads.append(o_ext[:, :DH] / o_ext[:, DH:DH + 1])
        z_parts.append(jnp.concatenate(heads, axis=1))
    z = z_parts[0] + z_parts[1] + z_parts[2]
    z = jnp.dot(z, fc1w_ref[...], preferred_element_type=jnp.float32) + fc1b_ref[...]
    z = jnp.where(z >= 0.0, z, 0.1 * z)
    z = jnp.dot(z, fc2w_ref[...], preferred_element_type=jnp.float32) + fc2b_ref[...]
    out_ref[...] = jnp.where(z >= 0.0, z, 0.1 * z)


def kernel(x, node_types, adj_mat_control, adj_mat_data, adj_mat_call,
           W_inst, W_var, W_const, a_src, a_dst, fc1_w, fc1_b, fc2_w, fc2_b):
    nt = node_types.astype(jnp.int32).reshape(N, 1)

    hd_idx = jnp.arange(H1) // DH
    col = jnp.arange(NCOL)
    sel = (hd_idx[:, None] == (col[None, :] % HEADS)).astype(jnp.float32) * LOG2E
    a_src_m = a_src.transpose(1, 2, 0).reshape(H1, 3)
    a_dst_m = a_dst.transpose(1, 2, 0).reshape(H1, 3)
    A_src = a_src_m[:, col // HEADS] * sel
    A_dst_T = (a_dst_m[:, col // HEADS] * sel).T

    hext, s, dT = pl.pallas_call(
        _proj_kernel,
        grid=(N // BLK,),
        in_specs=[
            pl.BlockSpec((BLK, D), lambda i: (i, 0)),
            pl.BlockSpec((BLK, 1), lambda i: (i, 0)),
            pl.BlockSpec((D, H1), lambda i: (0, 0)),
            pl.BlockSpec((D, H1), lambda i: (0, 0)),
            pl.BlockSpec((D, H1), lambda i: (0, 0)),
            pl.BlockSpec((H1, NCOL), lambda i: (0, 0)),
            pl.BlockSpec((NCOL, H1), lambda i: (0, 0)),
        ],
        out_specs=[
            pl.BlockSpec((BLK, HEADS * EXT), lambda i: (i, 0)),
            pl.BlockSpec((BLK, NCOL), lambda i: (i, 0)),
            pl.BlockSpec((NCOL, BLK), lambda i: (0, i)),
        ],
        out_shape=[
            jax.ShapeDtypeStruct((N, HEADS * EXT), jnp.bfloat16),
            jax.ShapeDtypeStruct((N, NCOL), jnp.float32),
            jax.ShapeDtypeStruct((NCOL, N), jnp.float32),
        ],
    )(x, nt, W_inst, W_var, W_const, A_src, A_dst_T)

    out = pl.pallas_call(
        _attn_kernel,
        grid=(N // BLK,),
        in_specs=[
            pl.BlockSpec((N, HEADS * EXT), lambda i: (0, 0)),
            pl.BlockSpec((BLK, NCOL), lambda i: (i, 0)),
            pl.BlockSpec((NCOL, N), lambda i: (0, 0)),
            pl.BlockSpec((BLK, N), lambda i: (i, 0)),
            pl.BlockSpec((BLK, N), lambda i: (i, 0)),
            pl.BlockSpec((BLK, N), lambda i: (i, 0)),
            pl.BlockSpec((H1, H2), lambda i: (0, 0)),
            pl.BlockSpec((1, H2), lambda i: (0, 0)),
            pl.BlockSpec((H2, NOUT), lambda i: (0, 0)),
            pl.BlockSpec((1, NOUT), lambda i: (0, 0)),
        ],
        out_specs=pl.BlockSpec((BLK, NOUT), lambda i: (i, 0)),
        out_shape=jax.ShapeDtypeStruct((N, NOUT), jnp.float32),
    )(hext, s, dT, adj_mat_control, adj_mat_data, adj_mat_call,
      fc1_w, fc1_b.reshape(1, H2), fc2_w, fc2_b.reshape(1, NOUT))
    return out
